# baseline (speedup 1.0000x reference)
import functools

import jax
import jax.numpy as jnp
from jax import lax
from jax.experimental import pallas as pl
from jax.experimental.pallas import tpu as pltpu
from jax.experimental.pallas import tpu_sc as plsc

N = 10000
D = 128
E = 320000

NC = 2
NS = 16
NW = NC * NS
CHUNK = 128
C = 79
EPAD = NW * C * CHUNK
DUMMY = N
NPAD = 10240
RPT = NPAD // NS

_BN_SCALE = 1.0 / (1.0 + 1e-5) ** 0.5
_LN_EPS = 1e-5


def _zero_vmem_rows(buf, nrows, ncols16):
    def zrow(i, _):
        for k in range(ncols16):
            buf[i, pl.ds(k * 16, 16)] = jnp.zeros((16,), jnp.float32)
        return 0
    lax.fori_loop(0, nrows, zrow, 0)


def _deg_body(dst_hbm, out_hbm, dst_v, buf, acc_sh):
    c = lax.axis_index("c")
    s = lax.axis_index("s")
    pltpu.sync_copy(dst_hbm.at[c, s], dst_v)
    _zero_vmem_rows(buf, CHUNK, D // 16)
    for b in range(RPT // CHUNK):
        pltpu.sync_copy(buf, acc_sh.at[pl.ds(s * RPT + b * CHUNK, CHUNK)])
    def orow(i, _):
        for k in range(D // 16):
            buf[i, pl.ds(k * 16, 16)] = jnp.ones((16,), jnp.float32)
        return 0
    lax.fori_loop(0, CHUNK, orow, 0)
    plsc.subcore_barrier()
    def body(j, _):
        pltpu.sync_copy(buf, acc_sh.at[dst_v.at[j]], add=True)
        return 0
    lax.fori_loop(0, C, body, 0)
    plsc.subcore_barrier()
    pltpu.sync_copy(acc_sh.at[pl.ds(s * RPT, RPT)],
                    out_hbm.at[c, pl.ds(s * RPT, RPT)])


def _sc_degree(dst_idx):
    mesh = plsc.VectorSubcoreMesh(core_axis_name="c", subcore_axis_name="s")
    return pl.kernel(
        _deg_body,
        out_type=jax.ShapeDtypeStruct((NC, NPAD, D), jnp.float32),
        mesh=mesh,
        scratch_types=[
            pltpu.VMEM((C, CHUNK), jnp.int32),
            pltpu.VMEM((CHUNK, D), jnp.float32),
            pltpu.VMEM_SHARED((NPAD, D), jnp.float32),
        ],
    )(dst_idx)


NBUF = 2
G = 16
NBLK = C // G
GB = CHUNK * D * 4


C0 = 40
C1 = 117
CMAX = max(C0, C1)
EPAD2 = NS * (C0 + C1) * CHUNK


def _prop_body(g_hbm, src_hbm, dst_hbm, out_hbm, src_v, dst_v, gbuf, acc_sh,
               sem):
    c = lax.axis_index("c")
    s = lax.axis_index("s")
    pltpu.sync_copy(src_hbm.at[c, s], src_v)
    pltpu.sync_copy(dst_hbm.at[c, s], dst_v)
    _zero_vmem_rows(gbuf, CHUNK, D // 16)
    for b in range(RPT // CHUNK):
        pltpu.sync_copy(gbuf, acc_sh.at[pl.ds(s * RPT + b * CHUNK, CHUNK)])
    plsc.subcore_barrier()
    nchunks = jnp.where(c == 0, C0, C1)
    def body(j, _):
        pltpu.async_copy(g_hbm.at[src_v.at[j]], gbuf, sem).wait()
        pltpu.sync_copy(gbuf, acc_sh.at[dst_v.at[j]], add=True)
        return 0
    lax.fori_loop(0, nchunks, body, 0)
    plsc.subcore_barrier()
    pltpu.sync_copy(acc_sh.at[pl.ds(s * RPT, RPT)],
                    out_hbm.at[c, pl.ds(s * RPT, RPT)])


def _sc_propagate(g, src_idx, dst_idx):
    mesh = plsc.VectorSubcoreMesh(core_axis_name="c", subcore_axis_name="s")
    return pl.kernel(
        _prop_body,
        out_type=jax.ShapeDtypeStruct((NC, NPAD, D), jnp.float32),
        mesh=mesh,
        scratch_types=[
            pltpu.VMEM((CMAX, CHUNK), jnp.int32),
            pltpu.VMEM((CMAX, CHUNK), jnp.int32),
            pltpu.VMEM((CHUNK, D), jnp.float32),
            pltpu.VMEM_SHARED((NPAD, D), jnp.float32),
            pltpu.SemaphoreType.DMA,
        ],
    )(g, src_idx, dst_idx)


BLK = 1024
BLKP = 1000


def _pre_body(x_ref, w_ref, d0_ref, d1_ref, g_ref, dinv_ref):
    deg = d0_ref[:, 0:1] + d1_ref[:, 0:1] + 1.0
    dinv = lax.rsqrt(deg)
    g_ref[...] = jnp.dot(x_ref[...], w_ref[...],
                         preferred_element_type=jnp.float32) * dinv
    dinv_ref[...] = jnp.broadcast_to(dinv, (BLK, 16))


def _tc_pre(xp, W1, deg0, deg1):
    grid = NPAD // BLK
    return pl.pallas_call(
        _pre_body,
        grid=(grid,),
        in_specs=[
            pl.BlockSpec((BLK, D), lambda i: (i, 0)),
            pl.BlockSpec((D, D), lambda i: (0, 0)),
            pl.BlockSpec((BLK, D), lambda i: (i, 0)),
            pl.BlockSpec((BLK, D), lambda i: (i, 0)),
        ],
        out_specs=[
            pl.BlockSpec((BLK, D), lambda i: (i, 0)),
            pl.BlockSpec((BLK, 16), lambda i: (i, 0)),
        ],
        out_shape=[
            jax.ShapeDtypeStruct((NPAD, D), jnp.float32),
            jax.ShapeDtypeStruct((NPAD, 16), jnp.float32),
        ],
    )(xp, W1, deg0, deg1)


def _mid_body(p0_ref, p1_ref, g_ref, dinv_ref, b_ref, bng_ref, bnb_ref, w_ref,
              out_ref):
    dv = dinv_ref[:, 0:1]
    pre = (p0_ref[...] + p1_ref[...] + g_ref[...]) * dv + b_ref[...]
    h = jnp.maximum(pre * _BN_SCALE * bng_ref[...] + bnb_ref[...], 0.0)
    out_ref[...] = jnp.dot(h, w_ref[...],
                           preferred_element_type=jnp.float32) * dv


def _tc_mid(p0, p1, g, dinv16, b, bng, bnb, Wn):
    grid = NPAD // BLK
    return pl.pallas_call(
        _mid_body,
        grid=(grid,),
        in_specs=[
            pl.BlockSpec((BLK, D), lambda i: (i, 0)),
            pl.BlockSpec((BLK, D), lambda i: (i, 0)),
            pl.BlockSpec((BLK, D), lambda i: (i, 0)),
            pl.BlockSpec((BLK, 16), lambda i: (i, 0)),
            pl.BlockSpec((1, D), lambda i: (0, 0)),
            pl.BlockSpec((1, D), lambda i: (0, 0)),
            pl.BlockSpec((1, D), lambda i: (0, 0)),
            pl.BlockSpec((D, D), lambda i: (0, 0)),
        ],
        out_specs=pl.BlockSpec((BLK, D), lambda i: (i, 0)),
        out_shape=jax.ShapeDtypeStruct((NPAD, D), jnp.float32),
    )(p0, p1, g, dinv16, b, bng, bnb, Wn)


def _post_body(p0_ref, p1_ref, g_ref, dinv_ref, b_ref, lng_ref, lnb_ref,
               emb_ref, pool_ref):
    i = pl.program_id(0)
    ngrid = pl.num_programs(0)
    dv = dinv_ref[:, 0:1]
    h = (p0_ref[...] + p1_ref[...] + g_ref[...]) * dv + b_ref[...]
    mu = jnp.mean(h, axis=-1, keepdims=True)
    var = jnp.mean((h - mu) ** 2, axis=-1, keepdims=True)
    e = (h - mu) * lax.rsqrt(var + _LN_EPS) * lng_ref[...] + lnb_ref[...]
    emb_ref[...] = e
    bsum = jnp.sum(e, axis=0, keepdims=True)

    @pl.when(i == 0)
    def _():
        pool_ref[...] = jnp.zeros_like(pool_ref)

    pool_ref[...] += bsum

    @pl.when(i == ngrid - 1)
    def _():
        pool_ref[...] = pool_ref[...] * (1.0 / N)


def _tc_post(p0, p1, g, dinv16, b, lng, lnb):
    grid = N // BLKP
    return pl.pallas_call(
        _post_body,
        grid=(grid,),
        in_specs=[
            pl.BlockSpec((BLKP, D), lambda i: (i, 0)),
            pl.BlockSpec((BLKP, D), lambda i: (i, 0)),
            pl.BlockSpec((BLKP, D), lambda i: (i, 0)),
            pl.BlockSpec((BLKP, 16), lambda i: (i, 0)),
            pl.BlockSpec((1, D), lambda i: (0, 0)),
            pl.BlockSpec((1, D), lambda i: (0, 0)),
            pl.BlockSpec((1, D), lambda i: (0, 0)),
        ],
        out_specs=[
            pl.BlockSpec((BLKP, D), lambda i: (i, 0)),
            pl.BlockSpec((1, D), lambda i: (0, 0)),
        ],
        out_shape=[
            jax.ShapeDtypeStruct((N, D), jnp.float32),
            jax.ShapeDtypeStruct((1, D), jnp.float32),
        ],
    )(p0, p1, g, dinv16, b, lng, lnb)


def kernel(x, edge_index, W1, b1, W2, b2, W3, b3, bn1_g, bn1_b, bn2_g, bn2_b,
           ln_g, ln_b):
    xp = jnp.zeros((NPAD, D), jnp.float32).at[:N].set(x)
    pad = EPAD - E
    src = jnp.concatenate(
        [edge_index[0], jnp.full((pad,), DUMMY, jnp.int32)]
    ).reshape(NC, NS, C, CHUNK)
    dst = jnp.concatenate(
        [edge_index[1], jnp.full((pad,), DUMMY, jnp.int32)]
    ).reshape(NC, NS, C, CHUNK)

    def split_uneven(v):
        e0n = NS * C0 * CHUNK
        vf = jnp.concatenate(
            [v, jnp.full((EPAD2 - E,), DUMMY, jnp.int32)])
        p0 = jnp.full((NS, CMAX, CHUNK), DUMMY, jnp.int32).at[:, :C0].set(
            vf[:e0n].reshape(NS, C0, CHUNK))
        p1 = vf[e0n:].reshape(NS, C1, CHUNK)
        return jnp.stack([p0, p1])

    src2 = split_uneven(edge_index[0])
    dst2 = split_uneven(edge_index[1])

    degp = _sc_degree(dst)
    g1, dinv16 = _tc_pre(xp, W1, degp[0], degp[1])

    s1 = _sc_propagate(g1, src2, dst2)
    g2 = _tc_mid(s1[0], s1[1], g1, dinv16, b1.reshape(1, D),
                 bn1_g.reshape(1, D), bn1_b.reshape(1, D), W2)

    s2 = _sc_propagate(g2, src2, dst2)
    g3 = _tc_mid(s2[0], s2[1], g2, dinv16, b2.reshape(1, D),
                 bn2_g.reshape(1, D), bn2_b.reshape(1, D), W3)

    s3 = _sc_propagate(g3, src2, dst2)
    node_embeddings, graph_embedding = _tc_post(
        s3[0], s3[1], g3, dinv16, b3.reshape(1, D),
        ln_g.reshape(1, D), ln_b.reshape(1, D))

    return (node_embeddings, graph_embedding)


# speedup vs baseline: 1.1335x; 1.1335x over previous
import functools

import jax
import jax.numpy as jnp
from jax import lax
from jax.experimental import pallas as pl
from jax.experimental.pallas import tpu as pltpu
from jax.experimental.pallas import tpu_sc as plsc

N = 10000
D = 128
E = 320000

NC = 2
NS = 16
NW = NC * NS
CHUNK = 128
C = 79
EPAD = NW * C * CHUNK
DUMMY = N
NPAD = 10240
RPT = NPAD // NS

_BN_SCALE = 1.0 / (1.0 + 1e-5) ** 0.5
_LN_EPS = 1e-5


def _zero_vmem_rows(buf, nrows, ncols16):
    def zrow(i, _):
        for k in range(ncols16):
            buf[i, pl.ds(k * 16, 16)] = jnp.zeros((16,), jnp.float32)
        return 0
    lax.fori_loop(0, nrows, zrow, 0)


def _deg_body(dst_hbm, out_hbm, dst_v, buf, acc_sh):
    c = lax.axis_index("c")
    s = lax.axis_index("s")
    pltpu.sync_copy(dst_hbm.at[c, s], dst_v)
    _zero_vmem_rows(buf, CHUNK, D // 16)
    for b in range(RPT // CHUNK):
        pltpu.sync_copy(buf, acc_sh.at[pl.ds(s * RPT + b * CHUNK, CHUNK)])
    def orow(i, _):
        for k in range(D // 16):
            buf[i, pl.ds(k * 16, 16)] = jnp.ones((16,), jnp.float32)
        return 0
    lax.fori_loop(0, CHUNK, orow, 0)
    plsc.subcore_barrier()
    def body(j, _):
        pltpu.sync_copy(buf, acc_sh.at[dst_v.at[j]], add=True)
        return 0
    lax.fori_loop(0, C, body, 0)
    plsc.subcore_barrier()
    pltpu.sync_copy(acc_sh.at[pl.ds(s * RPT, RPT)],
                    out_hbm.at[c, pl.ds(s * RPT, RPT)])


def _sc_degree(dst_idx):
    mesh = plsc.VectorSubcoreMesh(core_axis_name="c", subcore_axis_name="s")
    return pl.kernel(
        _deg_body,
        out_type=jax.ShapeDtypeStruct((NC, NPAD, D), jnp.float32),
        mesh=mesh,
        scratch_types=[
            pltpu.VMEM((C, CHUNK), jnp.int32),
            pltpu.VMEM((CHUNK, D), jnp.float32),
            pltpu.VMEM_SHARED((NPAD, D), jnp.float32),
        ],
    )(dst_idx)


NBUF = 2
G = 16
NBLK = C // G
GB = CHUNK * D * 4


C0 = 62
C1 = 95
CMAX = max(C0, C1)
EPAD2 = NS * (C0 + C1) * CHUNK


def _prop_body(g_hbm, src_hbm, dst_hbm, out_hbm, src_v, dst_v, gbuf, acc_sh,
               sem):
    c = lax.axis_index("c")
    s = lax.axis_index("s")
    pltpu.sync_copy(src_hbm.at[c, s], src_v)
    pltpu.sync_copy(dst_hbm.at[c, s], dst_v)
    _zero_vmem_rows(gbuf, CHUNK, D // 16)
    for b in range(RPT // CHUNK):
        pltpu.sync_copy(gbuf, acc_sh.at[pl.ds(s * RPT + b * CHUNK, CHUNK)])
    plsc.subcore_barrier()
    nchunks = jnp.where(c == 0, C0, C1)
    def body(j, _):
        pltpu.async_copy(g_hbm.at[src_v.at[j]], gbuf, sem).wait()
        pltpu.sync_copy(gbuf, acc_sh.at[dst_v.at[j]], add=True)
        return 0
    lax.fori_loop(0, nchunks, body, 0)
    plsc.subcore_barrier()
    pltpu.sync_copy(acc_sh.at[pl.ds(s * RPT, RPT)],
                    out_hbm.at[c, pl.ds(s * RPT, RPT)])


def _sc_propagate(g, src_idx, dst_idx):
    mesh = plsc.VectorSubcoreMesh(core_axis_name="c", subcore_axis_name="s")
    return pl.kernel(
        _prop_body,
        out_type=jax.ShapeDtypeStruct((NC, NPAD, D), jnp.float32),
        mesh=mesh,
        scratch_types=[
            pltpu.VMEM((CMAX, CHUNK), jnp.int32),
            pltpu.VMEM((CMAX, CHUNK), jnp.int32),
            pltpu.VMEM((CHUNK, D), jnp.float32),
            pltpu.VMEM_SHARED((NPAD, D), jnp.float32),
            pltpu.SemaphoreType.DMA,
        ],
    )(g, src_idx, dst_idx)


BLK = 1024
BLKP = 1000


def _pre_body(x_ref, w_ref, d0_ref, d1_ref, g_ref, dinv_ref):
    deg = d0_ref[:, 0:1] + d1_ref[:, 0:1] + 1.0
    dinv = lax.rsqrt(deg)
    g_ref[...] = jnp.dot(x_ref[...], w_ref[...],
                         preferred_element_type=jnp.float32) * dinv
    dinv_ref[...] = jnp.broadcast_to(dinv, (BLK, 16))


def _tc_pre(xp, W1, deg0, deg1):
    grid = NPAD // BLK
    return pl.pallas_call(
        _pre_body,
        grid=(grid,),
        in_specs=[
            pl.BlockSpec((BLK, D), lambda i: (i, 0)),
            pl.BlockSpec((D, D), lambda i: (0, 0)),
            pl.BlockSpec((BLK, D), lambda i: (i, 0)),
            pl.BlockSpec((BLK, D), lambda i: (i, 0)),
        ],
        out_specs=[
            pl.BlockSpec((BLK, D), lambda i: (i, 0)),
            pl.BlockSpec((BLK, 16), lambda i: (i, 0)),
        ],
        out_shape=[
            jax.ShapeDtypeStruct((NPAD, D), jnp.float32),
            jax.ShapeDtypeStruct((NPAD, 16), jnp.float32),
        ],
    )(xp, W1, deg0, deg1)


def _mid_body(p0_ref, p1_ref, g_ref, dinv_ref, b_ref, bng_ref, bnb_ref, w_ref,
              out_ref):
    dv = dinv_ref[:, 0:1]
    pre = (p0_ref[...] + p1_ref[...] + g_ref[...]) * dv + b_ref[...]
    h = jnp.maximum(pre * _BN_SCALE * bng_ref[...] + bnb_ref[...], 0.0)
    out_ref[...] = jnp.dot(h, w_ref[...],
                           preferred_element_type=jnp.float32) * dv


def _tc_mid(p0, p1, g, dinv16, b, bng, bnb, Wn):
    grid = NPAD // BLK
    return pl.pallas_call(
        _mid_body,
        grid=(grid,),
        in_specs=[
            pl.BlockSpec((BLK, D), lambda i: (i, 0)),
            pl.BlockSpec((BLK, D), lambda i: (i, 0)),
            pl.BlockSpec((BLK, D), lambda i: (i, 0)),
            pl.BlockSpec((BLK, 16), lambda i: (i, 0)),
            pl.BlockSpec((1, D), lambda i: (0, 0)),
            pl.BlockSpec((1, D), lambda i: (0, 0)),
            pl.BlockSpec((1, D), lambda i: (0, 0)),
            pl.BlockSpec((D, D), lambda i: (0, 0)),
        ],
        out_specs=pl.BlockSpec((BLK, D), lambda i: (i, 0)),
        out_shape=jax.ShapeDtypeStruct((NPAD, D), jnp.float32),
    )(p0, p1, g, dinv16, b, bng, bnb, Wn)


def _post_body(p0_ref, p1_ref, g_ref, dinv_ref, b_ref, lng_ref, lnb_ref,
               emb_ref, pool_ref):
    i = pl.program_id(0)
    ngrid = pl.num_programs(0)
    dv = dinv_ref[:, 0:1]
    h = (p0_ref[...] + p1_ref[...] + g_ref[...]) * dv + b_ref[...]
    mu = jnp.mean(h, axis=-1, keepdims=True)
    var = jnp.mean((h - mu) ** 2, axis=-1, keepdims=True)
    e = (h - mu) * lax.rsqrt(var + _LN_EPS) * lng_ref[...] + lnb_ref[...]
    emb_ref[...] = e
    bsum = jnp.sum(e, axis=0, keepdims=True)

    @pl.when(i == 0)
    def _():
        pool_ref[...] = jnp.zeros_like(pool_ref)

    pool_ref[...] += bsum

    @pl.when(i == ngrid - 1)
    def _():
        pool_ref[...] = pool_ref[...] * (1.0 / N)


def _tc_post(p0, p1, g, dinv16, b, lng, lnb):
    grid = N // BLKP
    return pl.pallas_call(
        _post_body,
        grid=(grid,),
        in_specs=[
            pl.BlockSpec((BLKP, D), lambda i: (i, 0)),
            pl.BlockSpec((BLKP, D), lambda i: (i, 0)),
            pl.BlockSpec((BLKP, D), lambda i: (i, 0)),
            pl.BlockSpec((BLKP, 16), lambda i: (i, 0)),
            pl.BlockSpec((1, D), lambda i: (0, 0)),
            pl.BlockSpec((1, D), lambda i: (0, 0)),
            pl.BlockSpec((1, D), lambda i: (0, 0)),
        ],
        out_specs=[
            pl.BlockSpec((BLKP, D), lambda i: (i, 0)),
            pl.BlockSpec((1, D), lambda i: (0, 0)),
        ],
        out_shape=[
            jax.ShapeDtypeStruct((N, D), jnp.float32),
            jax.ShapeDtypeStruct((1, D), jnp.float32),
        ],
    )(p0, p1, g, dinv16, b, lng, lnb)


def kernel(x, edge_index, W1, b1, W2, b2, W3, b3, bn1_g, bn1_b, bn2_g, bn2_b,
           ln_g, ln_b):
    xp = jnp.zeros((NPAD, D), jnp.float32).at[:N].set(x)
    pad = EPAD - E
    src = jnp.concatenate(
        [edge_index[0], jnp.full((pad,), DUMMY, jnp.int32)]
    ).reshape(NC, NS, C, CHUNK)
    dst = jnp.concatenate(
        [edge_index[1], jnp.full((pad,), DUMMY, jnp.int32)]
    ).reshape(NC, NS, C, CHUNK)

    def split_uneven(v):
        e0n = NS * C0 * CHUNK
        vf = jnp.concatenate(
            [v, jnp.full((EPAD2 - E,), DUMMY, jnp.int32)])
        p0 = jnp.full((NS, CMAX, CHUNK), DUMMY, jnp.int32).at[:, :C0].set(
            vf[:e0n].reshape(NS, C0, CHUNK))
        p1 = vf[e0n:].reshape(NS, C1, CHUNK)
        return jnp.stack([p0, p1])

    src2 = split_uneven(edge_index[0])
    dst2 = split_uneven(edge_index[1])

    degp = _sc_degree(dst)
    g1, dinv16 = _tc_pre(xp, W1, degp[0], degp[1])

    s1 = _sc_propagate(g1, src2, dst2)
    g2 = _tc_mid(s1[0], s1[1], g1, dinv16, b1.reshape(1, D),
                 bn1_g.reshape(1, D), bn1_b.reshape(1, D), W2)

    s2 = _sc_propagate(g2, src2, dst2)
    g3 = _tc_mid(s2[0], s2[1], g2, dinv16, b2.reshape(1, D),
                 bn2_g.reshape(1, D), bn2_b.reshape(1, D), W3)

    s3 = _sc_propagate(g3, src2, dst2)
    node_embeddings, graph_embedding = _tc_post(
        s3[0], s3[1], g3, dinv16, b3.reshape(1, D),
        ln_g.reshape(1, D), ln_b.reshape(1, D))

    return (node_embeddings, graph_embedding)


# speedup vs baseline: 1.1401x; 1.0058x over previous
import functools

import jax
import jax.numpy as jnp
from jax import lax
from jax.experimental import pallas as pl
from jax.experimental.pallas import tpu as pltpu
from jax.experimental.pallas import tpu_sc as plsc

N = 10000
D = 128
E = 320000

NC = 2
NS = 16
NW = NC * NS
CHUNK = 128
C = 79
EPAD = NW * C * CHUNK
DUMMY = N
NPAD = 10240
RPT = NPAD // NS

_BN_SCALE = 1.0 / (1.0 + 1e-5) ** 0.5
_LN_EPS = 1e-5


def _zero_vmem_rows(buf, nrows, ncols16):
    def zrow(i, _):
        for k in range(ncols16):
            buf[i, pl.ds(k * 16, 16)] = jnp.zeros((16,), jnp.float32)
        return 0
    lax.fori_loop(0, nrows, zrow, 0)


def _deg_body(dst_hbm, out_hbm, dst_v, buf, acc_sh):
    c = lax.axis_index("c")
    s = lax.axis_index("s")
    pltpu.sync_copy(dst_hbm.at[c, s], dst_v)
    _zero_vmem_rows(buf, CHUNK, D // 16)
    for b in range(RPT // CHUNK):
        pltpu.sync_copy(buf, acc_sh.at[pl.ds(s * RPT + b * CHUNK, CHUNK)])
    def orow(i, _):
        for k in range(D // 16):
            buf[i, pl.ds(k * 16, 16)] = jnp.ones((16,), jnp.float32)
        return 0
    lax.fori_loop(0, CHUNK, orow, 0)
    plsc.subcore_barrier()
    def body(j, _):
        pltpu.sync_copy(buf, acc_sh.at[dst_v.at[j]], add=True)
        return 0
    lax.fori_loop(0, C, body, 0)
    plsc.subcore_barrier()
    pltpu.sync_copy(acc_sh.at[pl.ds(s * RPT, RPT)],
                    out_hbm.at[c, pl.ds(s * RPT, RPT)])


def _sc_degree(dst_idx):
    mesh = plsc.VectorSubcoreMesh(core_axis_name="c", subcore_axis_name="s")
    return pl.kernel(
        _deg_body,
        out_type=jax.ShapeDtypeStruct((NC, NPAD, D), jnp.float32),
        mesh=mesh,
        scratch_types=[
            pltpu.VMEM((C, CHUNK), jnp.int32),
            pltpu.VMEM((CHUNK, D), jnp.float32),
            pltpu.VMEM_SHARED((NPAD, D), jnp.float32),
        ],
    )(dst_idx)


NBUF = 2
G = 16
NBLK = C // G
GB = CHUNK * D * 4


C0 = 70
C1 = 87
CMAX = max(C0, C1)
EPAD2 = NS * (C0 + C1) * CHUNK


def _prop_body(g_hbm, src_hbm, dst_hbm, out_hbm, src_v, dst_v, gbuf, acc_sh,
               sem):
    c = lax.axis_index("c")
    s = lax.axis_index("s")
    pltpu.sync_copy(src_hbm.at[c, s], src_v)
    pltpu.sync_copy(dst_hbm.at[c, s], dst_v)
    _zero_vmem_rows(gbuf, CHUNK, D // 16)
    for b in range(RPT // CHUNK):
        pltpu.sync_copy(gbuf, acc_sh.at[pl.ds(s * RPT + b * CHUNK, CHUNK)])
    plsc.subcore_barrier()
    nchunks = jnp.where(c == 0, C0, C1)
    def body(j, _):
        pltpu.async_copy(g_hbm.at[src_v.at[j]], gbuf, sem).wait()
        pltpu.sync_copy(gbuf, acc_sh.at[dst_v.at[j]], add=True)
        return 0
    lax.fori_loop(0, nchunks, body, 0)
    plsc.subcore_barrier()
    pltpu.sync_copy(acc_sh.at[pl.ds(s * RPT, RPT)],
                    out_hbm.at[c, pl.ds(s * RPT, RPT)])


def _sc_propagate(g, src_idx, dst_idx):
    mesh = plsc.VectorSubcoreMesh(core_axis_name="c", subcore_axis_name="s")
    return pl.kernel(
        _prop_body,
        out_type=jax.ShapeDtypeStruct((NC, NPAD, D), jnp.float32),
        mesh=mesh,
        scratch_types=[
            pltpu.VMEM((CMAX, CHUNK), jnp.int32),
            pltpu.VMEM((CMAX, CHUNK), jnp.int32),
            pltpu.VMEM((CHUNK, D), jnp.float32),
            pltpu.VMEM_SHARED((NPAD, D), jnp.float32),
            pltpu.SemaphoreType.DMA,
        ],
    )(g, src_idx, dst_idx)


BLK = 1024
BLKP = 1000


def _pre_body(x_ref, w_ref, d0_ref, d1_ref, g_ref, dinv_ref):
    deg = d0_ref[:, 0:1] + d1_ref[:, 0:1] + 1.0
    dinv = lax.rsqrt(deg)
    g_ref[...] = jnp.dot(x_ref[...], w_ref[...],
                         preferred_element_type=jnp.float32) * dinv
    dinv_ref[...] = jnp.broadcast_to(dinv, (BLK, 16))


def _tc_pre(xp, W1, deg0, deg1):
    grid = NPAD // BLK
    return pl.pallas_call(
        _pre_body,
        grid=(grid,),
        in_specs=[
            pl.BlockSpec((BLK, D), lambda i: (i, 0)),
            pl.BlockSpec((D, D), lambda i: (0, 0)),
            pl.BlockSpec((BLK, D), lambda i: (i, 0)),
            pl.BlockSpec((BLK, D), lambda i: (i, 0)),
        ],
        out_specs=[
            pl.BlockSpec((BLK, D), lambda i: (i, 0)),
            pl.BlockSpec((BLK, 16), lambda i: (i, 0)),
        ],
        out_shape=[
            jax.ShapeDtypeStruct((NPAD, D), jnp.float32),
            jax.ShapeDtypeStruct((NPAD, 16), jnp.float32),
        ],
    )(xp, W1, deg0, deg1)


def _mid_body(p0_ref, p1_ref, g_ref, dinv_ref, b_ref, bng_ref, bnb_ref, w_ref,
              out_ref):
    dv = dinv_ref[:, 0:1]
    pre = (p0_ref[...] + p1_ref[...] + g_ref[...]) * dv + b_ref[...]
    h = jnp.maximum(pre * _BN_SCALE * bng_ref[...] + bnb_ref[...], 0.0)
    out_ref[...] = jnp.dot(h, w_ref[...],
                           preferred_element_type=jnp.float32) * dv


def _tc_mid(p0, p1, g, dinv16, b, bng, bnb, Wn):
    grid = NPAD // BLK
    return pl.pallas_call(
        _mid_body,
        grid=(grid,),
        in_specs=[
            pl.BlockSpec((BLK, D), lambda i: (i, 0)),
            pl.BlockSpec((BLK, D), lambda i: (i, 0)),
            pl.BlockSpec((BLK, D), lambda i: (i, 0)),
            pl.BlockSpec((BLK, 16), lambda i: (i, 0)),
            pl.BlockSpec((1, D), lambda i: (0, 0)),
            pl.BlockSpec((1, D), lambda i: (0, 0)),
            pl.BlockSpec((1, D), lambda i: (0, 0)),
            pl.BlockSpec((D, D), lambda i: (0, 0)),
        ],
        out_specs=pl.BlockSpec((BLK, D), lambda i: (i, 0)),
        out_shape=jax.ShapeDtypeStruct((NPAD, D), jnp.float32),
    )(p0, p1, g, dinv16, b, bng, bnb, Wn)


def _post_body(p0_ref, p1_ref, g_ref, dinv_ref, b_ref, lng_ref, lnb_ref,
               emb_ref, pool_ref):
    i = pl.program_id(0)
    ngrid = pl.num_programs(0)
    dv = dinv_ref[:, 0:1]
    h = (p0_ref[...] + p1_ref[...] + g_ref[...]) * dv + b_ref[...]
    mu = jnp.mean(h, axis=-1, keepdims=True)
    var = jnp.mean((h - mu) ** 2, axis=-1, keepdims=True)
    e = (h - mu) * lax.rsqrt(var + _LN_EPS) * lng_ref[...] + lnb_ref[...]
    emb_ref[...] = e
    bsum = jnp.sum(e, axis=0, keepdims=True)

    @pl.when(i == 0)
    def _():
        pool_ref[...] = jnp.zeros_like(pool_ref)

    pool_ref[...] += bsum

    @pl.when(i == ngrid - 1)
    def _():
        pool_ref[...] = pool_ref[...] * (1.0 / N)


def _tc_post(p0, p1, g, dinv16, b, lng, lnb):
    grid = N // BLKP
    return pl.pallas_call(
        _post_body,
        grid=(grid,),
        in_specs=[
            pl.BlockSpec((BLKP, D), lambda i: (i, 0)),
            pl.BlockSpec((BLKP, D), lambda i: (i, 0)),
            pl.BlockSpec((BLKP, D), lambda i: (i, 0)),
            pl.BlockSpec((BLKP, 16), lambda i: (i, 0)),
            pl.BlockSpec((1, D), lambda i: (0, 0)),
            pl.BlockSpec((1, D), lambda i: (0, 0)),
            pl.BlockSpec((1, D), lambda i: (0, 0)),
        ],
        out_specs=[
            pl.BlockSpec((BLKP, D), lambda i: (i, 0)),
            pl.BlockSpec((1, D), lambda i: (0, 0)),
        ],
        out_shape=[
            jax.ShapeDtypeStruct((N, D), jnp.float32),
            jax.ShapeDtypeStruct((1, D), jnp.float32),
        ],
    )(p0, p1, g, dinv16, b, lng, lnb)


def kernel(x, edge_index, W1, b1, W2, b2, W3, b3, bn1_g, bn1_b, bn2_g, bn2_b,
           ln_g, ln_b):
    xp = jnp.zeros((NPAD, D), jnp.float32).at[:N].set(x)
    pad = EPAD - E
    src = jnp.concatenate(
        [edge_index[0], jnp.full((pad,), DUMMY, jnp.int32)]
    ).reshape(NC, NS, C, CHUNK)
    dst = jnp.concatenate(
        [edge_index[1], jnp.full((pad,), DUMMY, jnp.int32)]
    ).reshape(NC, NS, C, CHUNK)

    def split_uneven(v):
        e0n = NS * C0 * CHUNK
        vf = jnp.concatenate(
            [v, jnp.full((EPAD2 - E,), DUMMY, jnp.int32)])
        p0 = jnp.full((NS, CMAX, CHUNK), DUMMY, jnp.int32).at[:, :C0].set(
            vf[:e0n].reshape(NS, C0, CHUNK))
        p1 = vf[e0n:].reshape(NS, C1, CHUNK)
        return jnp.stack([p0, p1])

    src2 = split_uneven(edge_index[0])
    dst2 = split_uneven(edge_index[1])

    degp = _sc_degree(dst)
    g1, dinv16 = _tc_pre(xp, W1, degp[0], degp[1])

    s1 = _sc_propagate(g1, src2, dst2)
    g2 = _tc_mid(s1[0], s1[1], g1, dinv16, b1.reshape(1, D),
                 bn1_g.reshape(1, D), bn1_b.reshape(1, D), W2)

    s2 = _sc_propagate(g2, src2, dst2)
    g3 = _tc_mid(s2[0], s2[1], g2, dinv16, b2.reshape(1, D),
                 bn2_g.reshape(1, D), bn2_b.reshape(1, D), W3)

    s3 = _sc_propagate(g3, src2, dst2)
    node_embeddings, graph_embedding = _tc_post(
        s3[0], s3[1], g3, dinv16, b3.reshape(1, D),
        ln_g.reshape(1, D), ln_b.reshape(1, D))

    return (node_embeddings, graph_embedding)


# speedup vs baseline: 1.1669x; 1.0235x over previous
import functools

import jax
import jax.numpy as jnp
from jax import lax
from jax.experimental import pallas as pl
from jax.experimental.pallas import tpu as pltpu
from jax.experimental.pallas import tpu_sc as plsc

N = 10000
D = 128
E = 320000

NC = 2
NS = 16
NW = NC * NS
CHUNK = 128
C = 79
EPAD = NW * C * CHUNK
DUMMY = N
NPAD = 10240
RPT = NPAD // NS

_BN_SCALE = 1.0 / (1.0 + 1e-5) ** 0.5
_LN_EPS = 1e-5


def _zero_vmem_rows(buf, nrows, ncols16):
    def zrow(i, _):
        for k in range(ncols16):
            buf[i, pl.ds(k * 16, 16)] = jnp.zeros((16,), jnp.float32)
        return 0
    lax.fori_loop(0, nrows, zrow, 0)


def _deg_body(dst_hbm, out_hbm, dst_v, buf, acc_sh):
    c = lax.axis_index("c")
    s = lax.axis_index("s")
    pltpu.sync_copy(dst_hbm.at[c, s], dst_v)
    _zero_vmem_rows(buf, CHUNK, D // 16)
    for b in range(RPT // CHUNK):
        pltpu.sync_copy(buf, acc_sh.at[pl.ds(s * RPT + b * CHUNK, CHUNK)])
    def orow(i, _):
        for k in range(D // 16):
            buf[i, pl.ds(k * 16, 16)] = jnp.ones((16,), jnp.float32)
        return 0
    lax.fori_loop(0, CHUNK, orow, 0)
    plsc.subcore_barrier()
    def body(j, _):
        pltpu.sync_copy(buf, acc_sh.at[dst_v.at[j]], add=True)
        return 0
    lax.fori_loop(0, C, body, 0)
    plsc.subcore_barrier()
    pltpu.sync_copy(acc_sh.at[pl.ds(s * RPT, RPT)],
                    out_hbm.at[c, pl.ds(s * RPT, RPT)])


def _sc_degree(dst_idx):
    mesh = plsc.VectorSubcoreMesh(core_axis_name="c", subcore_axis_name="s")
    return pl.kernel(
        _deg_body,
        out_type=jax.ShapeDtypeStruct((NC, NPAD, D), jnp.float32),
        mesh=mesh,
        scratch_types=[
            pltpu.VMEM((C, CHUNK), jnp.int32),
            pltpu.VMEM((CHUNK, D), jnp.float32),
            pltpu.VMEM_SHARED((NPAD, D), jnp.float32),
        ],
    )(dst_idx)


NBUF = 2
G = 16
NBLK = C // G
GB = CHUNK * D * 4


C0 = 74
C1 = 83
CMAX = max(C0, C1)
EPAD2 = NS * (C0 + C1) * CHUNK


def _prop_body(g_hbm, src_hbm, dst_hbm, out_hbm, src_v, dst_v, gbuf, acc_sh,
               sem):
    c = lax.axis_index("c")
    s = lax.axis_index("s")
    pltpu.sync_copy(src_hbm.at[c, s], src_v)
    pltpu.sync_copy(dst_hbm.at[c, s], dst_v)
    _zero_vmem_rows(gbuf, CHUNK, D // 16)
    for b in range(RPT // CHUNK):
        pltpu.sync_copy(gbuf, acc_sh.at[pl.ds(s * RPT + b * CHUNK, CHUNK)])
    plsc.subcore_barrier()
    nchunks = jnp.where(c == 0, C0, C1)
    def body(j, _):
        pltpu.async_copy(g_hbm.at[src_v.at[j]], gbuf, sem).wait()
        pltpu.sync_copy(gbuf, acc_sh.at[dst_v.at[j]], add=True)
        return 0
    lax.fori_loop(0, nchunks, body, 0)
    plsc.subcore_barrier()
    pltpu.sync_copy(acc_sh.at[pl.ds(s * RPT, RPT)],
                    out_hbm.at[c, pl.ds(s * RPT, RPT)])


def _sc_propagate(g, src_idx, dst_idx):
    mesh = plsc.VectorSubcoreMesh(core_axis_name="c", subcore_axis_name="s")
    return pl.kernel(
        _prop_body,
        out_type=jax.ShapeDtypeStruct((NC, NPAD, D), jnp.float32),
        mesh=mesh,
        scratch_types=[
            pltpu.VMEM((CMAX, CHUNK), jnp.int32),
            pltpu.VMEM((CMAX, CHUNK), jnp.int32),
            pltpu.VMEM((CHUNK, D), jnp.float32),
            pltpu.VMEM_SHARED((NPAD, D), jnp.float32),
            pltpu.SemaphoreType.DMA,
        ],
    )(g, src_idx, dst_idx)


BLK = 1024
BLKP = 1000


def _pre_body(x_ref, w_ref, d0_ref, d1_ref, g_ref, dinv_ref):
    deg = d0_ref[:, 0:1] + d1_ref[:, 0:1] + 1.0
    dinv = lax.rsqrt(deg)
    g_ref[...] = jnp.dot(x_ref[...], w_ref[...],
                         preferred_element_type=jnp.float32) * dinv
    dinv_ref[...] = jnp.broadcast_to(dinv, (BLK, 16))


def _tc_pre(xp, W1, deg0, deg1):
    grid = NPAD // BLK
    return pl.pallas_call(
        _pre_body,
        grid=(grid,),
        in_specs=[
            pl.BlockSpec((BLK, D), lambda i: (i, 0)),
            pl.BlockSpec((D, D), lambda i: (0, 0)),
            pl.BlockSpec((BLK, D), lambda i: (i, 0)),
            pl.BlockSpec((BLK, D), lambda i: (i, 0)),
        ],
        out_specs=[
            pl.BlockSpec((BLK, D), lambda i: (i, 0)),
            pl.BlockSpec((BLK, 16), lambda i: (i, 0)),
        ],
        out_shape=[
            jax.ShapeDtypeStruct((NPAD, D), jnp.float32),
            jax.ShapeDtypeStruct((NPAD, 16), jnp.float32),
        ],
    )(xp, W1, deg0, deg1)


def _mid_body(p0_ref, p1_ref, g_ref, dinv_ref, b_ref, bng_ref, bnb_ref, w_ref,
              out_ref):
    dv = dinv_ref[:, 0:1]
    pre = (p0_ref[...] + p1_ref[...] + g_ref[...]) * dv + b_ref[...]
    h = jnp.maximum(pre * _BN_SCALE * bng_ref[...] + bnb_ref[...], 0.0)
    out_ref[...] = jnp.dot(h, w_ref[...],
                           preferred_element_type=jnp.float32) * dv


def _tc_mid(p0, p1, g, dinv16, b, bng, bnb, Wn):
    grid = NPAD // BLK
    return pl.pallas_call(
        _mid_body,
        grid=(grid,),
        in_specs=[
            pl.BlockSpec((BLK, D), lambda i: (i, 0)),
            pl.BlockSpec((BLK, D), lambda i: (i, 0)),
            pl.BlockSpec((BLK, D), lambda i: (i, 0)),
            pl.BlockSpec((BLK, 16), lambda i: (i, 0)),
            pl.BlockSpec((1, D), lambda i: (0, 0)),
            pl.BlockSpec((1, D), lambda i: (0, 0)),
            pl.BlockSpec((1, D), lambda i: (0, 0)),
            pl.BlockSpec((D, D), lambda i: (0, 0)),
        ],
        out_specs=pl.BlockSpec((BLK, D), lambda i: (i, 0)),
        out_shape=jax.ShapeDtypeStruct((NPAD, D), jnp.float32),
    )(p0, p1, g, dinv16, b, bng, bnb, Wn)


def _post_body(p0_ref, p1_ref, g_ref, dinv_ref, b_ref, lng_ref, lnb_ref,
               emb_ref, pool_ref):
    i = pl.program_id(0)
    ngrid = pl.num_programs(0)
    dv = dinv_ref[:, 0:1]
    h = (p0_ref[...] + p1_ref[...] + g_ref[...]) * dv + b_ref[...]
    mu = jnp.mean(h, axis=-1, keepdims=True)
    var = jnp.mean((h - mu) ** 2, axis=-1, keepdims=True)
    e = (h - mu) * lax.rsqrt(var + _LN_EPS) * lng_ref[...] + lnb_ref[...]
    emb_ref[...] = e
    bsum = jnp.sum(e, axis=0, keepdims=True)

    @pl.when(i == 0)
    def _():
        pool_ref[...] = jnp.zeros_like(pool_ref)

    pool_ref[...] += bsum

    @pl.when(i == ngrid - 1)
    def _():
        pool_ref[...] = pool_ref[...] * (1.0 / N)


def _tc_post(p0, p1, g, dinv16, b, lng, lnb):
    grid = N // BLKP
    return pl.pallas_call(
        _post_body,
        grid=(grid,),
        in_specs=[
            pl.BlockSpec((BLKP, D), lambda i: (i, 0)),
            pl.BlockSpec((BLKP, D), lambda i: (i, 0)),
            pl.BlockSpec((BLKP, D), lambda i: (i, 0)),
            pl.BlockSpec((BLKP, 16), lambda i: (i, 0)),
            pl.BlockSpec((1, D), lambda i: (0, 0)),
            pl.BlockSpec((1, D), lambda i: (0, 0)),
            pl.BlockSpec((1, D), lambda i: (0, 0)),
        ],
        out_specs=[
            pl.BlockSpec((BLKP, D), lambda i: (i, 0)),
            pl.BlockSpec((1, D), lambda i: (0, 0)),
        ],
        out_shape=[
            jax.ShapeDtypeStruct((N, D), jnp.float32),
            jax.ShapeDtypeStruct((1, D), jnp.float32),
        ],
    )(p0, p1, g, dinv16, b, lng, lnb)


def kernel(x, edge_index, W1, b1, W2, b2, W3, b3, bn1_g, bn1_b, bn2_g, bn2_b,
           ln_g, ln_b):
    xp = jnp.zeros((NPAD, D), jnp.float32).at[:N].set(x)
    pad = EPAD - E
    src = jnp.concatenate(
        [edge_index[0], jnp.full((pad,), DUMMY, jnp.int32)]
    ).reshape(NC, NS, C, CHUNK)
    dst = jnp.concatenate(
        [edge_index[1], jnp.full((pad,), DUMMY, jnp.int32)]
    ).reshape(NC, NS, C, CHUNK)

    def split_uneven(v):
        e0n = NS * C0 * CHUNK
        vf = jnp.concatenate(
            [v, jnp.full((EPAD2 - E,), DUMMY, jnp.int32)])
        p0 = jnp.full((NS, CMAX, CHUNK), DUMMY, jnp.int32).at[:, :C0].set(
            vf[:e0n].reshape(NS, C0, CHUNK))
        p1 = vf[e0n:].reshape(NS, C1, CHUNK)
        return jnp.stack([p0, p1])

    src2 = split_uneven(edge_index[0])
    dst2 = split_uneven(edge_index[1])

    degp = _sc_degree(dst)
    g1, dinv16 = _tc_pre(xp, W1, degp[0], degp[1])

    s1 = _sc_propagate(g1, src2, dst2)
    g2 = _tc_mid(s1[0], s1[1], g1, dinv16, b1.reshape(1, D),
                 bn1_g.reshape(1, D), bn1_b.reshape(1, D), W2)

    s2 = _sc_propagate(g2, src2, dst2)
    g3 = _tc_mid(s2[0], s2[1], g2, dinv16, b2.reshape(1, D),
                 bn2_g.reshape(1, D), bn2_b.reshape(1, D), W3)

    s3 = _sc_propagate(g3, src2, dst2)
    node_embeddings, graph_embedding = _tc_post(
        s3[0], s3[1], g3, dinv16, b3.reshape(1, D),
        ln_g.reshape(1, D), ln_b.reshape(1, D))

    return (node_embeddings, graph_embedding)


# speedup vs baseline: 1.1806x; 1.0117x over previous
import functools

import jax
import jax.numpy as jnp
from jax import lax
from jax.experimental import pallas as pl
from jax.experimental.pallas import tpu as pltpu
from jax.experimental.pallas import tpu_sc as plsc

N = 10000
D = 128
E = 320000

NC = 2
NS = 16
NW = NC * NS
CHUNK = 128
C = 79
EPAD = NW * C * CHUNK
DUMMY = N
NPAD = 10240
RPT = NPAD // NS

_BN_SCALE = 1.0 / (1.0 + 1e-5) ** 0.5
_LN_EPS = 1e-5


def _zero_vmem_rows(buf, nrows, ncols16):
    def zrow(i, _):
        for k in range(ncols16):
            buf[i, pl.ds(k * 16, 16)] = jnp.zeros((16,), jnp.float32)
        return 0
    lax.fori_loop(0, nrows, zrow, 0)


def _deg_body(dst_hbm, out_hbm, dst_v, buf, acc_sh):
    c = lax.axis_index("c")
    s = lax.axis_index("s")
    pltpu.sync_copy(dst_hbm.at[c, s], dst_v)
    _zero_vmem_rows(buf, CHUNK, D // 16)
    for b in range(RPT // CHUNK):
        pltpu.sync_copy(buf, acc_sh.at[pl.ds(s * RPT + b * CHUNK, CHUNK)])
    def orow(i, _):
        for k in range(D // 16):
            buf[i, pl.ds(k * 16, 16)] = jnp.ones((16,), jnp.float32)
        return 0
    lax.fori_loop(0, CHUNK, orow, 0)
    plsc.subcore_barrier()
    def body(j, _):
        pltpu.sync_copy(buf, acc_sh.at[dst_v.at[j]], add=True)
        return 0
    lax.fori_loop(0, C, body, 0)
    plsc.subcore_barrier()
    pltpu.sync_copy(acc_sh.at[pl.ds(s * RPT, RPT)],
                    out_hbm.at[c, pl.ds(s * RPT, RPT)])


def _sc_degree(dst_idx):
    mesh = plsc.VectorSubcoreMesh(core_axis_name="c", subcore_axis_name="s")
    return pl.kernel(
        _deg_body,
        out_type=jax.ShapeDtypeStruct((NC, NPAD, D), jnp.float32),
        mesh=mesh,
        scratch_types=[
            pltpu.VMEM((C, CHUNK), jnp.int32),
            pltpu.VMEM((CHUNK, D), jnp.float32),
            pltpu.VMEM_SHARED((NPAD, D), jnp.float32),
        ],
    )(dst_idx)


NBUF = 2
G = 16
NBLK = C // G
GB = CHUNK * D * 4


C0 = 76
C1 = 81
CMAX = max(C0, C1)
EPAD2 = NS * (C0 + C1) * CHUNK


def _prop_body(g_hbm, src_hbm, dst_hbm, out_hbm, src_v, dst_v, gbuf, acc_sh,
               sem):
    c = lax.axis_index("c")
    s = lax.axis_index("s")
    pltpu.sync_copy(src_hbm.at[c, s], src_v)
    pltpu.sync_copy(dst_hbm.at[c, s], dst_v)
    _zero_vmem_rows(gbuf, CHUNK, D // 16)
    for b in range(RPT // CHUNK):
        pltpu.sync_copy(gbuf, acc_sh.at[pl.ds(s * RPT + b * CHUNK, CHUNK)])
    plsc.subcore_barrier()
    nchunks = jnp.where(c == 0, C0, C1)
    def body(j, _):
        pltpu.async_copy(g_hbm.at[src_v.at[j]], gbuf, sem).wait()
        pltpu.sync_copy(gbuf, acc_sh.at[dst_v.at[j]], add=True)
        return 0
    lax.fori_loop(0, nchunks, body, 0)
    plsc.subcore_barrier()
    pltpu.sync_copy(acc_sh.at[pl.ds(s * RPT, RPT)],
                    out_hbm.at[c, pl.ds(s * RPT, RPT)])


def _sc_propagate(g, src_idx, dst_idx):
    mesh = plsc.VectorSubcoreMesh(core_axis_name="c", subcore_axis_name="s")
    return pl.kernel(
        _prop_body,
        out_type=jax.ShapeDtypeStruct((NC, NPAD, D), jnp.float32),
        mesh=mesh,
        scratch_types=[
            pltpu.VMEM((CMAX, CHUNK), jnp.int32),
            pltpu.VMEM((CMAX, CHUNK), jnp.int32),
            pltpu.VMEM((CHUNK, D), jnp.float32),
            pltpu.VMEM_SHARED((NPAD, D), jnp.float32),
            pltpu.SemaphoreType.DMA,
        ],
    )(g, src_idx, dst_idx)


BLK = 1024
BLKP = 1000


def _pre_body(x_ref, w_ref, d0_ref, d1_ref, g_ref, dinv_ref):
    deg = d0_ref[:, 0:1] + d1_ref[:, 0:1] + 1.0
    dinv = lax.rsqrt(deg)
    g_ref[...] = jnp.dot(x_ref[...], w_ref[...],
                         preferred_element_type=jnp.float32) * dinv
    dinv_ref[...] = jnp.broadcast_to(dinv, (BLK, 16))


def _tc_pre(xp, W1, deg0, deg1):
    grid = NPAD // BLK
    return pl.pallas_call(
        _pre_body,
        grid=(grid,),
        in_specs=[
            pl.BlockSpec((BLK, D), lambda i: (i, 0)),
            pl.BlockSpec((D, D), lambda i: (0, 0)),
            pl.BlockSpec((BLK, D), lambda i: (i, 0)),
            pl.BlockSpec((BLK, D), lambda i: (i, 0)),
        ],
        out_specs=[
            pl.BlockSpec((BLK, D), lambda i: (i, 0)),
            pl.BlockSpec((BLK, 16), lambda i: (i, 0)),
        ],
        out_shape=[
            jax.ShapeDtypeStruct((NPAD, D), jnp.float32),
            jax.ShapeDtypeStruct((NPAD, 16), jnp.float32),
        ],
    )(xp, W1, deg0, deg1)


def _mid_body(p0_ref, p1_ref, g_ref, dinv_ref, b_ref, bng_ref, bnb_ref, w_ref,
              out_ref):
    dv = dinv_ref[:, 0:1]
    pre = (p0_ref[...] + p1_ref[...] + g_ref[...]) * dv + b_ref[...]
    h = jnp.maximum(pre * _BN_SCALE * bng_ref[...] + bnb_ref[...], 0.0)
    out_ref[...] = jnp.dot(h, w_ref[...],
                           preferred_element_type=jnp.float32) * dv


def _tc_mid(p0, p1, g, dinv16, b, bng, bnb, Wn):
    grid = NPAD // BLK
    return pl.pallas_call(
        _mid_body,
        grid=(grid,),
        in_specs=[
            pl.BlockSpec((BLK, D), lambda i: (i, 0)),
            pl.BlockSpec((BLK, D), lambda i: (i, 0)),
            pl.BlockSpec((BLK, D), lambda i: (i, 0)),
            pl.BlockSpec((BLK, 16), lambda i: (i, 0)),
            pl.BlockSpec((1, D), lambda i: (0, 0)),
            pl.BlockSpec((1, D), lambda i: (0, 0)),
            pl.BlockSpec((1, D), lambda i: (0, 0)),
            pl.BlockSpec((D, D), lambda i: (0, 0)),
        ],
        out_specs=pl.BlockSpec((BLK, D), lambda i: (i, 0)),
        out_shape=jax.ShapeDtypeStruct((NPAD, D), jnp.float32),
    )(p0, p1, g, dinv16, b, bng, bnb, Wn)


def _post_body(p0_ref, p1_ref, g_ref, dinv_ref, b_ref, lng_ref, lnb_ref,
               emb_ref, pool_ref):
    i = pl.program_id(0)
    ngrid = pl.num_programs(0)
    dv = dinv_ref[:, 0:1]
    h = (p0_ref[...] + p1_ref[...] + g_ref[...]) * dv + b_ref[...]
    mu = jnp.mean(h, axis=-1, keepdims=True)
    var = jnp.mean((h - mu) ** 2, axis=-1, keepdims=True)
    e = (h - mu) * lax.rsqrt(var + _LN_EPS) * lng_ref[...] + lnb_ref[...]
    emb_ref[...] = e
    bsum = jnp.sum(e, axis=0, keepdims=True)

    @pl.when(i == 0)
    def _():
        pool_ref[...] = jnp.zeros_like(pool_ref)

    pool_ref[...] += bsum

    @pl.when(i == ngrid - 1)
    def _():
        pool_ref[...] = pool_ref[...] * (1.0 / N)


def _tc_post(p0, p1, g, dinv16, b, lng, lnb):
    grid = N // BLKP
    return pl.pallas_call(
        _post_body,
        grid=(grid,),
        in_specs=[
            pl.BlockSpec((BLKP, D), lambda i: (i, 0)),
            pl.BlockSpec((BLKP, D), lambda i: (i, 0)),
            pl.BlockSpec((BLKP, D), lambda i: (i, 0)),
            pl.BlockSpec((BLKP, 16), lambda i: (i, 0)),
            pl.BlockSpec((1, D), lambda i: (0, 0)),
            pl.BlockSpec((1, D), lambda i: (0, 0)),
            pl.BlockSpec((1, D), lambda i: (0, 0)),
        ],
        out_specs=[
            pl.BlockSpec((BLKP, D), lambda i: (i, 0)),
            pl.BlockSpec((1, D), lambda i: (0, 0)),
        ],
        out_shape=[
            jax.ShapeDtypeStruct((N, D), jnp.float32),
            jax.ShapeDtypeStruct((1, D), jnp.float32),
        ],
    )(p0, p1, g, dinv16, b, lng, lnb)


def kernel(x, edge_index, W1, b1, W2, b2, W3, b3, bn1_g, bn1_b, bn2_g, bn2_b,
           ln_g, ln_b):
    xp = jnp.zeros((NPAD, D), jnp.float32).at[:N].set(x)
    pad = EPAD - E
    src = jnp.concatenate(
        [edge_index[0], jnp.full((pad,), DUMMY, jnp.int32)]
    ).reshape(NC, NS, C, CHUNK)
    dst = jnp.concatenate(
        [edge_index[1], jnp.full((pad,), DUMMY, jnp.int32)]
    ).reshape(NC, NS, C, CHUNK)

    def split_uneven(v):
        e0n = NS * C0 * CHUNK
        vf = jnp.concatenate(
            [v, jnp.full((EPAD2 - E,), DUMMY, jnp.int32)])
        p0 = jnp.full((NS, CMAX, CHUNK), DUMMY, jnp.int32).at[:, :C0].set(
            vf[:e0n].reshape(NS, C0, CHUNK))
        p1 = vf[e0n:].reshape(NS, C1, CHUNK)
        return jnp.stack([p0, p1])

    src2 = split_uneven(edge_index[0])
    dst2 = split_uneven(edge_index[1])

    degp = _sc_degree(dst)
    g1, dinv16 = _tc_pre(xp, W1, degp[0], degp[1])

    s1 = _sc_propagate(g1, src2, dst2)
    g2 = _tc_mid(s1[0], s1[1], g1, dinv16, b1.reshape(1, D),
                 bn1_g.reshape(1, D), bn1_b.reshape(1, D), W2)

    s2 = _sc_propagate(g2, src2, dst2)
    g3 = _tc_mid(s2[0], s2[1], g2, dinv16, b2.reshape(1, D),
                 bn2_g.reshape(1, D), bn2_b.reshape(1, D), W3)

    s3 = _sc_propagate(g3, src2, dst2)
    node_embeddings, graph_embedding = _tc_post(
        s3[0], s3[1], g3, dinv16, b3.reshape(1, D),
        ln_g.reshape(1, D), ln_b.reshape(1, D))

    return (node_embeddings, graph_embedding)


# speedup vs baseline: 1.1909x; 1.0087x over previous
import functools

import jax
import jax.numpy as jnp
from jax import lax
from jax.experimental import pallas as pl
from jax.experimental.pallas import tpu as pltpu
from jax.experimental.pallas import tpu_sc as plsc

N = 10000
D = 128
E = 320000

NC = 2
NS = 16
NW = NC * NS
CHUNK = 128
C = 79
EPAD = NW * C * CHUNK
DUMMY = N
NPAD = 10240
RPT = NPAD // NS

_BN_SCALE = 1.0 / (1.0 + 1e-5) ** 0.5
_LN_EPS = 1e-5


def _zero_vmem_rows(buf, nrows, ncols16):
    def zrow(i, _):
        for k in range(ncols16):
            buf[i, pl.ds(k * 16, 16)] = jnp.zeros((16,), jnp.float32)
        return 0
    lax.fori_loop(0, nrows, zrow, 0)


def _deg_body(dst_hbm, out_hbm, dst_v, buf, acc_sh):
    c = lax.axis_index("c")
    s = lax.axis_index("s")
    pltpu.sync_copy(dst_hbm.at[c, s], dst_v)
    _zero_vmem_rows(buf, CHUNK, D // 16)
    for b in range(RPT // CHUNK):
        pltpu.sync_copy(buf, acc_sh.at[pl.ds(s * RPT + b * CHUNK, CHUNK)])
    def orow(i, _):
        for k in range(D // 16):
            buf[i, pl.ds(k * 16, 16)] = jnp.ones((16,), jnp.float32)
        return 0
    lax.fori_loop(0, CHUNK, orow, 0)
    plsc.subcore_barrier()
    def body(j, _):
        pltpu.sync_copy(buf, acc_sh.at[dst_v.at[j]], add=True)
        return 0
    lax.fori_loop(0, C, body, 0)
    plsc.subcore_barrier()
    pltpu.sync_copy(acc_sh.at[pl.ds(s * RPT, RPT)],
                    out_hbm.at[c, pl.ds(s * RPT, RPT)])


def _sc_degree(dst_idx):
    mesh = plsc.VectorSubcoreMesh(core_axis_name="c", subcore_axis_name="s")
    return pl.kernel(
        _deg_body,
        out_type=jax.ShapeDtypeStruct((NC, NPAD, D), jnp.float32),
        mesh=mesh,
        scratch_types=[
            pltpu.VMEM((C, CHUNK), jnp.int32),
            pltpu.VMEM((CHUNK, D), jnp.float32),
            pltpu.VMEM_SHARED((NPAD, D), jnp.float32),
        ],
    )(dst_idx)


NBUF = 2
G = 16
NBLK = C // G
GB = CHUNK * D * 4


C0 = 78
C1 = 79
CMAX = max(C0, C1)
EPAD2 = NS * (C0 + C1) * CHUNK


def _prop_body(g_hbm, src_hbm, dst_hbm, out_hbm, src_v, dst_v, gbuf, acc_sh,
               sem):
    c = lax.axis_index("c")
    s = lax.axis_index("s")
    pltpu.sync_copy(src_hbm.at[c, s], src_v)
    pltpu.sync_copy(dst_hbm.at[c, s], dst_v)
    _zero_vmem_rows(gbuf, CHUNK, D // 16)
    for b in range(RPT // CHUNK):
        pltpu.sync_copy(gbuf, acc_sh.at[pl.ds(s * RPT + b * CHUNK, CHUNK)])
    plsc.subcore_barrier()
    nchunks = jnp.where(c == 0, C0, C1)
    def body(j, _):
        pltpu.async_copy(g_hbm.at[src_v.at[j]], gbuf, sem).wait()
        pltpu.sync_copy(gbuf, acc_sh.at[dst_v.at[j]], add=True)
        return 0
    lax.fori_loop(0, nchunks, body, 0)
    plsc.subcore_barrier()
    pltpu.sync_copy(acc_sh.at[pl.ds(s * RPT, RPT)],
                    out_hbm.at[c, pl.ds(s * RPT, RPT)])


def _sc_propagate(g, src_idx, dst_idx):
    mesh = plsc.VectorSubcoreMesh(core_axis_name="c", subcore_axis_name="s")
    return pl.kernel(
        _prop_body,
        out_type=jax.ShapeDtypeStruct((NC, NPAD, D), jnp.float32),
        mesh=mesh,
        scratch_types=[
            pltpu.VMEM((CMAX, CHUNK), jnp.int32),
            pltpu.VMEM((CMAX, CHUNK), jnp.int32),
            pltpu.VMEM((CHUNK, D), jnp.float32),
            pltpu.VMEM_SHARED((NPAD, D), jnp.float32),
            pltpu.SemaphoreType.DMA,
        ],
    )(g, src_idx, dst_idx)


BLK = 1024
BLKP = 1000


def _pre_body(x_ref, w_ref, d0_ref, d1_ref, g_ref, dinv_ref):
    deg = d0_ref[:, 0:1] + d1_ref[:, 0:1] + 1.0
    dinv = lax.rsqrt(deg)
    g_ref[...] = jnp.dot(x_ref[...], w_ref[...],
                         preferred_element_type=jnp.float32) * dinv
    dinv_ref[...] = jnp.broadcast_to(dinv, (BLK, 16))


def _tc_pre(xp, W1, deg0, deg1):
    grid = NPAD // BLK
    return pl.pallas_call(
        _pre_body,
        grid=(grid,),
        in_specs=[
            pl.BlockSpec((BLK, D), lambda i: (i, 0)),
            pl.BlockSpec((D, D), lambda i: (0, 0)),
            pl.BlockSpec((BLK, D), lambda i: (i, 0)),
            pl.BlockSpec((BLK, D), lambda i: (i, 0)),
        ],
        out_specs=[
            pl.BlockSpec((BLK, D), lambda i: (i, 0)),
            pl.BlockSpec((BLK, 16), lambda i: (i, 0)),
        ],
        out_shape=[
            jax.ShapeDtypeStruct((NPAD, D), jnp.float32),
            jax.ShapeDtypeStruct((NPAD, 16), jnp.float32),
        ],
    )(xp, W1, deg0, deg1)


def _mid_body(p0_ref, p1_ref, g_ref, dinv_ref, b_ref, bng_ref, bnb_ref, w_ref,
              out_ref):
    dv = dinv_ref[:, 0:1]
    pre = (p0_ref[...] + p1_ref[...] + g_ref[...]) * dv + b_ref[...]
    h = jnp.maximum(pre * _BN_SCALE * bng_ref[...] + bnb_ref[...], 0.0)
    out_ref[...] = jnp.dot(h, w_ref[...],
                           preferred_element_type=jnp.float32) * dv


def _tc_mid(p0, p1, g, dinv16, b, bng, bnb, Wn):
    grid = NPAD // BLK
    return pl.pallas_call(
        _mid_body,
        grid=(grid,),
        in_specs=[
            pl.BlockSpec((BLK, D), lambda i: (i, 0)),
            pl.BlockSpec((BLK, D), lambda i: (i, 0)),
            pl.BlockSpec((BLK, D), lambda i: (i, 0)),
            pl.BlockSpec((BLK, 16), lambda i: (i, 0)),
            pl.BlockSpec((1, D), lambda i: (0, 0)),
            pl.BlockSpec((1, D), lambda i: (0, 0)),
            pl.BlockSpec((1, D), lambda i: (0, 0)),
            pl.BlockSpec((D, D), lambda i: (0, 0)),
        ],
        out_specs=pl.BlockSpec((BLK, D), lambda i: (i, 0)),
        out_shape=jax.ShapeDtypeStruct((NPAD, D), jnp.float32),
    )(p0, p1, g, dinv16, b, bng, bnb, Wn)


def _post_body(p0_ref, p1_ref, g_ref, dinv_ref, b_ref, lng_ref, lnb_ref,
               emb_ref, pool_ref):
    i = pl.program_id(0)
    ngrid = pl.num_programs(0)
    dv = dinv_ref[:, 0:1]
    h = (p0_ref[...] + p1_ref[...] + g_ref[...]) * dv + b_ref[...]
    mu = jnp.mean(h, axis=-1, keepdims=True)
    var = jnp.mean((h - mu) ** 2, axis=-1, keepdims=True)
    e = (h - mu) * lax.rsqrt(var + _LN_EPS) * lng_ref[...] + lnb_ref[...]
    emb_ref[...] = e
    bsum = jnp.sum(e, axis=0, keepdims=True)

    @pl.when(i == 0)
    def _():
        pool_ref[...] = jnp.zeros_like(pool_ref)

    pool_ref[...] += bsum

    @pl.when(i == ngrid - 1)
    def _():
        pool_ref[...] = pool_ref[...] * (1.0 / N)


def _tc_post(p0, p1, g, dinv16, b, lng, lnb):
    grid = N // BLKP
    return pl.pallas_call(
        _post_body,
        grid=(grid,),
        in_specs=[
            pl.BlockSpec((BLKP, D), lambda i: (i, 0)),
            pl.BlockSpec((BLKP, D), lambda i: (i, 0)),
            pl.BlockSpec((BLKP, D), lambda i: (i, 0)),
            pl.BlockSpec((BLKP, 16), lambda i: (i, 0)),
            pl.BlockSpec((1, D), lambda i: (0, 0)),
            pl.BlockSpec((1, D), lambda i: (0, 0)),
            pl.BlockSpec((1, D), lambda i: (0, 0)),
        ],
        out_specs=[
            pl.BlockSpec((BLKP, D), lambda i: (i, 0)),
            pl.BlockSpec((1, D), lambda i: (0, 0)),
        ],
        out_shape=[
            jax.ShapeDtypeStruct((N, D), jnp.float32),
            jax.ShapeDtypeStruct((1, D), jnp.float32),
        ],
    )(p0, p1, g, dinv16, b, lng, lnb)


def kernel(x, edge_index, W1, b1, W2, b2, W3, b3, bn1_g, bn1_b, bn2_g, bn2_b,
           ln_g, ln_b):
    xp = jnp.zeros((NPAD, D), jnp.float32).at[:N].set(x)
    pad = EPAD - E
    src = jnp.concatenate(
        [edge_index[0], jnp.full((pad,), DUMMY, jnp.int32)]
    ).reshape(NC, NS, C, CHUNK)
    dst = jnp.concatenate(
        [edge_index[1], jnp.full((pad,), DUMMY, jnp.int32)]
    ).reshape(NC, NS, C, CHUNK)

    def split_uneven(v):
        e0n = NS * C0 * CHUNK
        vf = jnp.concatenate(
            [v, jnp.full((EPAD2 - E,), DUMMY, jnp.int32)])
        p0 = jnp.full((NS, CMAX, CHUNK), DUMMY, jnp.int32).at[:, :C0].set(
            vf[:e0n].reshape(NS, C0, CHUNK))
        p1 = vf[e0n:].reshape(NS, C1, CHUNK)
        return jnp.stack([p0, p1])

    src2 = split_uneven(edge_index[0])
    dst2 = split_uneven(edge_index[1])

    degp = _sc_degree(dst)
    g1, dinv16 = _tc_pre(xp, W1, degp[0], degp[1])

    s1 = _sc_propagate(g1, src2, dst2)
    g2 = _tc_mid(s1[0], s1[1], g1, dinv16, b1.reshape(1, D),
                 bn1_g.reshape(1, D), bn1_b.reshape(1, D), W2)

    s2 = _sc_propagate(g2, src2, dst2)
    g3 = _tc_mid(s2[0], s2[1], g2, dinv16, b2.reshape(1, D),
                 bn2_g.reshape(1, D), bn2_b.reshape(1, D), W3)

    s3 = _sc_propagate(g3, src2, dst2)
    node_embeddings, graph_embedding = _tc_post(
        s3[0], s3[1], g3, dinv16, b3.reshape(1, D),
        ln_g.reshape(1, D), ln_b.reshape(1, D))

    return (node_embeddings, graph_embedding)
